# Initial kernel scaffold; baseline (speedup 1.0000x reference)
import jax
import jax.numpy as jnp
from jax.experimental import pallas as pl


def kernel(indice_sequence, embedding_matrix):
    raise NotImplementedError("write your pallas kernel here")


# speedup vs baseline: 1.1118x; 1.1118x over previous
import jax
import jax.numpy as jnp
from jax import lax
from jax.experimental import pallas as pl
from jax.experimental.pallas import tpu as pltpu
from jax.experimental.pallas import tpu_sc as plsc

VOCAB = 1000000
D_MODEL = 32
BATCH = 16384
HIST = 50

NC = 2
NS = 16
NW = NC * NS

B_FLAT = BATCH * HIST
B_PER_W = B_FLAT // NW
CHUNK = 1600
N_CHUNKS = B_PER_W // CHUNK


def _gather_body(idx_hbm, table_hbm, out_hbm, idx_v, rows_v, sems, osem):
    wid = lax.axis_index("s") * NC + lax.axis_index("c")
    base = wid * B_PER_W
    pltpu.sync_copy(idx_hbm.at[pl.ds(base, B_PER_W)], idx_v)

    def gather(g, buf):
        return pltpu.async_copy(
            table_hbm.at[idx_v.at[pl.ds(g * CHUNK, CHUNK)]],
            rows_v.at[buf],
            sems[buf],
        )

    def store(g, buf):
        return pltpu.async_copy(
            rows_v.at[buf],
            out_hbm.at[pl.ds(base + g * CHUNK, CHUNK)],
            osem,
        )

    pending = gather(0, 0)
    for g in range(1, N_CHUNKS):
        nxt = gather(g, g % 2)
        pending.wait()
        store(g - 1, (g - 1) % 2).wait()
        pending = nxt
    pending.wait()
    store(N_CHUNKS - 1, (N_CHUNKS - 1) % 2).wait()


@jax.jit
def _embed_gather(idx_flat, table):
    mesh = plsc.VectorSubcoreMesh(
        core_axis_name="c", subcore_axis_name="s", num_cores=NC, num_subcores=NS
    )
    return pl.kernel(
        _gather_body,
        out_type=jax.ShapeDtypeStruct((B_FLAT, D_MODEL), jnp.float32),
        mesh=mesh,
        compiler_params=pltpu.CompilerParams(use_tc_tiling_on_sc=False),
        scratch_types=[
            pltpu.VMEM((B_PER_W,), jnp.int32),
            pltpu.VMEM((2, CHUNK, D_MODEL), jnp.float32),
            (pltpu.SemaphoreType.DMA, pltpu.SemaphoreType.DMA),
            pltpu.SemaphoreType.DMA,
        ],
    )(idx_flat, table)


def kernel(indice_sequence, embedding_matrix):
    idx_flat = indice_sequence.reshape(-1).astype(jnp.int32)
    out = _embed_gather(idx_flat, embedding_matrix)
    return out.reshape(BATCH, HIST, D_MODEL)


# speedup vs baseline: 1.8038x; 1.6225x over previous
import jax
import jax.numpy as jnp
from jax import lax
from jax.experimental import pallas as pl
from jax.experimental.pallas import tpu as pltpu
from jax.experimental.pallas import tpu_sc as plsc

VOCAB = 1000000
D_MODEL = 32
BATCH = 16384
HIST = 50

NC = 2
NS = 16
NW = NC * NS

ROWS_PER_W = BATCH // NW
STAGE_ROWS = 16
N_STAGES = ROWS_PER_W // STAGE_ROWS


def _gather_body(idx_hbm, table_hbm, out_hbm, idx2d_v, stage_v, gsems, osems):
    wid = lax.axis_index("s") * NC + lax.axis_index("c")
    b0 = wid * ROWS_PER_W
    pltpu.sync_copy(idx_hbm.at[pl.ds(b0, ROWS_PER_W), :], idx2d_v)

    def issue_gathers(t, parity):
        for r in range(STAGE_ROWS):
            pltpu.async_copy(
                table_hbm.at[idx2d_v.at[t * STAGE_ROWS + r]],
                stage_v.at[parity * STAGE_ROWS + r],
                gsems[parity],
            )

    def drain(sem, parity):
        pltpu.make_async_copy(
            out_hbm.at[pl.ds(0, STAGE_ROWS)],
            stage_v.at[pl.ds(parity * STAGE_ROWS, STAGE_ROWS)],
            sem,
        ).wait()

    def store(t, parity):
        pltpu.async_copy(
            stage_v.at[pl.ds(parity * STAGE_ROWS, STAGE_ROWS)],
            out_hbm.at[pl.ds(b0 + t * STAGE_ROWS, STAGE_ROWS)],
            osems[parity],
        )

    issue_gathers(0, 0)
    issue_gathers(1, 1)
    drain(gsems[0], 0)
    store(0, 0)

    def step(k, carry):
        del carry
        t0 = 2 * k
        drain(osems[0], 0)
        issue_gathers(t0, 0)
        drain(gsems[1], 1)
        store(t0 - 1, 1)
        t1 = t0 + 1
        drain(osems[1], 1)
        issue_gathers(t1, 1)
        drain(gsems[0], 0)
        store(t1 - 1, 0)
        return 0

    lax.fori_loop(1, N_STAGES // 2, step, 0, unroll=False)
    drain(gsems[1], 1)
    store(N_STAGES - 1, 1)
    drain(osems[0], 0)
    drain(osems[1], 1)


@jax.jit
def _embed_gather(idx, table):
    mesh = plsc.VectorSubcoreMesh(
        core_axis_name="c", subcore_axis_name="s", num_cores=NC, num_subcores=NS
    )
    return pl.kernel(
        _gather_body,
        out_type=jax.ShapeDtypeStruct((BATCH, HIST, D_MODEL), jnp.float32),
        mesh=mesh,
        compiler_params=pltpu.CompilerParams(use_tc_tiling_on_sc=False),
        scratch_types=[
            pltpu.VMEM((ROWS_PER_W, HIST), jnp.int32),
            pltpu.VMEM((2 * STAGE_ROWS, HIST, D_MODEL), jnp.float32),
            (pltpu.SemaphoreType.DMA, pltpu.SemaphoreType.DMA),
            (pltpu.SemaphoreType.DMA, pltpu.SemaphoreType.DMA),
        ],
    )(idx, table)


def kernel(indice_sequence, embedding_matrix):
    return _embed_gather(indice_sequence.astype(jnp.int32), embedding_matrix)
